# Initial kernel scaffold; baseline (speedup 1.0000x reference)
import jax
import jax.numpy as jnp
from jax.experimental import pallas as pl


def kernel(x, edge_index, batch, W1a, b1a, W2a, b2a, W1b, b1b, W2b, b2b, Wl1, bl1, Wl2, bl2):
    raise NotImplementedError("write your pallas kernel here")


# speedup vs baseline: 2.6341x; 2.6341x over previous
import functools
import jax
import jax.numpy as jnp
from jax import lax
from jax.experimental import pallas as pl
from jax.experimental.pallas import tpu as pltpu
from jax.experimental.pallas import tpu_sc as plsc

N = 10000
E = 320000
NC = 2
NS = 16
NW = NC * NS
CH = 80
RT = 128
E_PAD = NW * RT * CH
IB = 32
OB = RT // IB
ACC_R = 10240
TRASH = ACC_R - 8
ZSPAN = ACC_R // NS
ZR = 128
WSPAN = 632
WLAST = N - 15 * WSPAN


def _hop_body(C, h_hbm, srcm_hbm, dstm_hbm, out_hbm, idx_s, idx_d, rows, zbuf, acc_sh):
    cid = lax.axis_index("c")
    sid = lax.axis_index("s")
    wid = sid * NC + cid

    @pl.loop(0, ZR)
    def _(i):
        @pl.loop(0, C // 16)
        def _(j):
            zbuf[i, pl.ds(j * 16, 16)] = jnp.zeros((16,), jnp.float32)

    @pl.loop(0, ZSPAN // ZR)
    def _(k):
        pltpu.sync_copy(zbuf, acc_sh.at[pl.ds(sid * ZSPAN + k * ZR, ZR)])

    plsc.subcore_barrier()

    row0 = wid * RT

    @pl.loop(0, OB)
    def _(ob):
        pltpu.sync_copy(srcm_hbm.at[pl.ds(row0 + ob * IB, IB)], idx_s)
        pltpu.sync_copy(dstm_hbm.at[pl.ds(row0 + ob * IB, IB)], idx_d)

        @pl.loop(0, IB)
        def _(j):
            pltpu.sync_copy(h_hbm.at[idx_s.at[j]], rows)
            pltpu.sync_copy(rows, acc_sh.at[idx_d.at[j]], add=True)

    plsc.subcore_barrier()

    @pl.when(sid < NS - 1)
    def _():
        pltpu.sync_copy(acc_sh.at[pl.ds(sid * WSPAN, WSPAN)],
                        out_hbm.at[cid, pl.ds(sid * WSPAN, WSPAN)])

    @pl.when(sid == NS - 1)
    def _():
        pltpu.sync_copy(acc_sh.at[pl.ds(15 * WSPAN, WLAST)],
                        out_hbm.at[cid, pl.ds(15 * WSPAN, WLAST)])


def _make_hop(C):
    mesh = plsc.VectorSubcoreMesh(core_axis_name="c", subcore_axis_name="s",
                                  num_cores=NC, num_subcores=NS)
    return pl.kernel(
        functools.partial(_hop_body, C),
        out_type=jax.ShapeDtypeStruct((NC, N, C), jnp.float32),
        mesh=mesh,
        scratch_types=[
            pltpu.VMEM((IB, CH), jnp.int32),
            pltpu.VMEM((IB, CH), jnp.int32),
            pltpu.VMEM((CH, C), jnp.float32),
            pltpu.VMEM((ZR, C), jnp.float32),
            pltpu.VMEM_SHARED((ACC_R, C), jnp.float32),
        ],
        name=f"sc_hop_c{C}",
    )


_hop128 = _make_hop(128)


def _dense1_body(a0, a1, x, w1, b1, w2, b2, out):
    s = a0[...] + a1[...]
    t = jnp.dot(s, w1[:128, :], preferred_element_type=jnp.float32)
    t += jnp.dot(x[...], w1[128:, :], preferred_element_type=jnp.float32)
    t = jnp.maximum(t + b1[...], 0.0)
    r = jnp.dot(t, w2[...], preferred_element_type=jnp.float32) + b2[...]
    out[...] = jnp.concatenate([r, jnp.zeros_like(r)], axis=1)


_dense1 = pl.pallas_call(
    _dense1_body,
    out_shape=jax.ShapeDtypeStruct((N, 128), jnp.float32),
)


def _combine_body(a0, a1, out):
    out[...] = a0[...] + a1[...]


_combine = pl.pallas_call(
    _combine_body,
    out_shape=jax.ShapeDtypeStruct((N, 128), jnp.float32),
)


def _dense2_head_body(a0, a1, h1, w1, b1, w2, b2, wl1, bl1, wl2, bl2, out):
    s = a0[...][:, :64] + a1[...][:, :64]
    t = jnp.dot(s, w1[:64, :], preferred_element_type=jnp.float32)
    t += jnp.dot(h1[...][:, :64], w1[64:, :], preferred_element_type=jnp.float32)
    t = jnp.maximum(t + b1[...], 0.0)
    u = jnp.dot(t, w2[...], preferred_element_type=jnp.float32) + b2[...]
    v = jnp.maximum(
        jnp.dot(u, wl1[...], preferred_element_type=jnp.float32) + bl1[...], 0.0)
    out[...] = jnp.dot(v, wl2[...], preferred_element_type=jnp.float32) + bl2[...]


_dense2_head = pl.pallas_call(
    _dense2_head_body,
    out_shape=jax.ShapeDtypeStruct((N, 7), jnp.float32),
)


def kernel(x, edge_index, batch,
           W1a, b1a, W2a, b2a,
           W1b, b1b, W2b, b2b,
           Wl1, bl1, Wl2, bl2):
    pad = E_PAD - E
    src_m = jnp.concatenate(
        [edge_index[0], jnp.zeros((pad,), jnp.int32)]).reshape(E_PAD // CH, CH)
    dst_m = jnp.concatenate(
        [edge_index[1], jnp.full((pad,), TRASH, jnp.int32)]).reshape(E_PAD // CH, CH)

    p = _hop128(x, src_m, dst_m)
    h1 = _dense1(p[0], p[1], x, W1a, b1a, W2a, b2a)

    h = h1
    for i in range(3):
        p = _hop128(h, src_m, dst_m)
        h = _combine(p[0], p[1])
    p = _hop128(h, src_m, dst_m)
    return _dense2_head(p[0], p[1], h1, W1b, b1b, W2b, b2b, Wl1, bl1, Wl2, bl2)
